# baseline (device time: 80559 ns/iter reference)
import jax
import jax.numpy as jnp
from jax import lax
from jax.experimental import pallas as pl
from jax.experimental.pallas import tpu as pltpu

N_DEV = 4


def kernel(x, w_mat):
    m_per, k = x.shape
    _, n_per = w_mat.shape
    m_half = m_per // 2
    m_seg = m_half // 2

    def body(x_hbm, w_hbm, out_hbm,
             x_vm, w_vm, out_vm, from_l, from_r, diag_t, diag_b,
             ld_sem, st_sem, p1_send, p1_recv, p2_send, p2_recv):
        my_pos = lax.axis_index("i")
        left = (my_pos - 1) % N_DEV
        right = (my_pos + 1) % N_DEV

        ld_x = pltpu.make_async_copy(x_hbm, x_vm, ld_sem.at[0])
        ld_w = pltpu.make_async_copy(w_hbm, w_vm, ld_sem.at[1])
        ld_x.start()
        ld_w.start()

        barrier_sem = pltpu.get_barrier_semaphore()
        for nbr in [left, right]:
            pl.semaphore_signal(
                barrier_sem, inc=1,
                device_id=(nbr,), device_id_type=pl.DeviceIdType.MESH,
            )
        pl.semaphore_wait(barrier_sem, 2)

        top = pl.ds(0, m_half)
        bot = pl.ds(m_half, m_half)

        def p1(dir_, seg_idx, rows):
            tgt = right if dir_ == 0 else left
            dst = from_l if dir_ == 0 else from_r
            return pltpu.make_async_remote_copy(
                src_ref=x_hbm.at[rows],
                dst_ref=dst.at[rows],
                send_sem=p1_send.at[dir_, seg_idx],
                recv_sem=p1_recv.at[dir_, seg_idx],
                device_id=(tgt,),
                device_id_type=pl.DeviceIdType.MESH,
            )

        p1cw_t = p1(0, 0, top)
        p1cw_b = p1(0, 1, bot)
        p1ccw_b = p1(1, 0, bot)
        p1ccw_t = p1(1, 1, top)

        p1cw_t.start()
        p1ccw_b.start()
        p1cw_b.start()
        p1ccw_t.start()

        def p2(dir_, seg_idx):
            if dir_ == 0:
                rows = pl.ds(seg_idx * m_seg, m_seg)
                return pltpu.make_async_remote_copy(
                    src_ref=from_l.at[rows],
                    dst_ref=diag_t.at[rows],
                    send_sem=p2_send.at[0, seg_idx],
                    recv_sem=p2_recv.at[0, seg_idx],
                    device_id=(right,),
                    device_id_type=pl.DeviceIdType.MESH,
                )
            rows = pl.ds(m_half + seg_idx * m_seg, m_seg)
            drows = pl.ds(seg_idx * m_seg, m_seg)
            return pltpu.make_async_remote_copy(
                src_ref=from_r.at[rows],
                dst_ref=diag_b.at[drows],
                send_sem=p2_send.at[1, seg_idx],
                recv_sem=p2_recv.at[1, seg_idx],
                device_id=(left,),
                device_id_type=pl.DeviceIdType.MESH,
            )

        p2cw = [p2(0, s) for s in range(2)]
        p2ccw = [p2(1, s) for s in range(2)]

        stores = []

        def gemm(rows_src, out_start, rows_n):
            out_vm[pl.ds(out_start, rows_n), :] = jnp.maximum(
                jnp.dot(rows_src, w_vm[...],
                        preferred_element_type=jnp.float32),
                0.0,
            )
            st = pltpu.make_async_copy(
                out_vm.at[pl.ds(out_start, rows_n)],
                out_hbm.at[pl.ds(out_start, rows_n)],
                st_sem.at[len(stores)],
            )
            st.start()
            stores.append(st)

        ld_w.wait()
        ld_x.wait()
        gemm(x_vm[...], my_pos * m_per, m_per)

        p1cw_t.wait_recv()
        p2cw[0].start()
        p2cw[1].start()
        p1ccw_b.wait_recv()
        p2ccw[0].start()
        p2ccw[1].start()

        gemm(from_l[top, :], left * m_per, m_half)
        gemm(from_r[bot, :], right * m_per + m_half, m_half)

        p1cw_b.wait_recv()
        gemm(from_l[bot, :], left * m_per + m_half, m_half)
        p1ccw_t.wait_recv()
        gemm(from_r[top, :], right * m_per, m_half)

        diag = (my_pos + 2) % N_DEV
        for s in range(2):
            p2cw[s].wait_recv()
            gemm(diag_t[s * m_seg:(s + 1) * m_seg, :],
                 diag * m_per + s * m_seg, m_seg)
            p2ccw[s].wait_recv()
            gemm(diag_b[s * m_seg:(s + 1) * m_seg, :],
                 diag * m_per + m_half + s * m_seg, m_seg)

        for st in stores:
            st.wait()
        for r in (p1cw_t, p1cw_b, p1ccw_b, p1ccw_t,
                  p2cw[0], p2cw[1], p2ccw[0], p2ccw[1]):
            r.wait_send()

    return pl.pallas_call(
        body,
        out_shape=jax.ShapeDtypeStruct((N_DEV * m_per, n_per), jnp.float32),
        in_specs=[
            pl.BlockSpec(memory_space=pltpu.MemorySpace.HBM),
            pl.BlockSpec(memory_space=pltpu.MemorySpace.HBM),
        ],
        out_specs=pl.BlockSpec(memory_space=pltpu.MemorySpace.HBM),
        scratch_shapes=[
            pltpu.VMEM((m_per, k), jnp.float32),
            pltpu.VMEM((k, n_per), jnp.float32),
            pltpu.VMEM((N_DEV * m_per, n_per), jnp.float32),
            pltpu.VMEM((m_per, k), jnp.float32),
            pltpu.VMEM((m_per, k), jnp.float32),
            pltpu.VMEM((m_half, k), jnp.float32),
            pltpu.VMEM((m_half, k), jnp.float32),
            pltpu.SemaphoreType.DMA((2,)),
            pltpu.SemaphoreType.DMA((9,)),
            pltpu.SemaphoreType.DMA((2, 2)),
            pltpu.SemaphoreType.DMA((2, 2)),
            pltpu.SemaphoreType.DMA((2, 2)),
            pltpu.SemaphoreType.DMA((2, 2)),
        ],
        compiler_params=pltpu.CompilerParams(collective_id=0),
    )(x, w_mat)


# device time: 46285 ns/iter; 1.7405x vs baseline; 1.7405x over previous
import jax
import jax.numpy as jnp
from jax import lax
from jax.experimental import pallas as pl
from jax.experimental.pallas import tpu as pltpu

N_DEV = 4


def kernel(x, w_mat):
    m_per, k = x.shape
    _, n_per = w_mat.shape
    m_half = m_per // 2
    m_seg = m_half // 2

    def body(x_ref, w_ref, out_ref,
             x_bf, from_l, from_r, diag_t, diag_b,
             p1_send, p1_recv, p2_send, p2_recv):
        my_pos = lax.axis_index("i")
        left = (my_pos - 1) % N_DEV
        right = (my_pos + 1) % N_DEV

        barrier_sem = pltpu.get_barrier_semaphore()
        for nbr in [left, right]:
            pl.semaphore_signal(
                barrier_sem, inc=1,
                device_id=(nbr,), device_id_type=pl.DeviceIdType.MESH,
            )
        pl.semaphore_wait(barrier_sem, 2)

        top = pl.ds(0, m_half)
        bot = pl.ds(m_half, m_half)

        x_bf[...] = x_ref[...].astype(jnp.bfloat16)

        def p1(dir_, seg_idx, rows):
            tgt = right if dir_ == 0 else left
            dst = from_l if dir_ == 0 else from_r
            return pltpu.make_async_remote_copy(
                src_ref=x_bf.at[rows],
                dst_ref=dst.at[rows],
                send_sem=p1_send.at[dir_, seg_idx],
                recv_sem=p1_recv.at[dir_, seg_idx],
                device_id=(tgt,),
                device_id_type=pl.DeviceIdType.MESH,
            )

        p1cw_t = p1(0, 0, top)
        p1cw_b = p1(0, 1, bot)
        p1ccw_b = p1(1, 0, bot)
        p1ccw_t = p1(1, 1, top)

        p1cw_t.start()
        p1ccw_b.start()
        p1cw_b.start()
        p1ccw_t.start()

        def p2(dir_, seg_idx):
            if dir_ == 0:
                rows = pl.ds(seg_idx * m_seg, m_seg)
                return pltpu.make_async_remote_copy(
                    src_ref=from_l.at[rows],
                    dst_ref=diag_t.at[rows],
                    send_sem=p2_send.at[0, seg_idx],
                    recv_sem=p2_recv.at[0, seg_idx],
                    device_id=(right,),
                    device_id_type=pl.DeviceIdType.MESH,
                )
            rows = pl.ds(m_half + seg_idx * m_seg, m_seg)
            drows = pl.ds(seg_idx * m_seg, m_seg)
            return pltpu.make_async_remote_copy(
                src_ref=from_r.at[rows],
                dst_ref=diag_b.at[drows],
                send_sem=p2_send.at[1, seg_idx],
                recv_sem=p2_recv.at[1, seg_idx],
                device_id=(left,),
                device_id_type=pl.DeviceIdType.MESH,
            )

        p2cw = [p2(0, s) for s in range(2)]
        p2ccw = [p2(1, s) for s in range(2)]

        def gemm(rows_src, out_start, rows_n):
            out_ref[pl.ds(out_start, rows_n), :] = jnp.maximum(
                jnp.dot(rows_src.astype(jnp.float32), w_ref[...],
                        preferred_element_type=jnp.float32),
                0.0,
            )

        gemm(x_ref[...], my_pos * m_per, m_per)

        p1cw_t.wait_recv()
        p2cw[0].start()
        p2cw[1].start()
        p1ccw_b.wait_recv()
        p2ccw[0].start()
        p2ccw[1].start()

        gemm(from_l[top, :], left * m_per, m_half)
        gemm(from_r[bot, :], right * m_per + m_half, m_half)

        p1cw_b.wait_recv()
        gemm(from_l[bot, :], left * m_per + m_half, m_half)
        p1ccw_t.wait_recv()
        gemm(from_r[top, :], right * m_per, m_half)

        diag = (my_pos + 2) % N_DEV
        for s in range(2):
            p2cw[s].wait_recv()
            gemm(diag_t[s * m_seg:(s + 1) * m_seg, :],
                 diag * m_per + s * m_seg, m_seg)
            p2ccw[s].wait_recv()
            gemm(diag_b[s * m_seg:(s + 1) * m_seg, :],
                 diag * m_per + m_half + s * m_seg, m_seg)

        for r in (p1cw_t, p1cw_b, p1ccw_b, p1ccw_t,
                  p2cw[0], p2cw[1], p2ccw[0], p2ccw[1]):
            r.wait_send()

    return pl.pallas_call(
        body,
        out_shape=jax.ShapeDtypeStruct((N_DEV * m_per, n_per), jnp.float32),
        in_specs=[
            pl.BlockSpec(memory_space=pltpu.VMEM),
            pl.BlockSpec(memory_space=pltpu.VMEM),
        ],
        out_specs=pl.BlockSpec(memory_space=pltpu.VMEM),
        scratch_shapes=[
            pltpu.VMEM((m_per, k), jnp.bfloat16),
            pltpu.VMEM((m_per, k), jnp.bfloat16),
            pltpu.VMEM((m_per, k), jnp.bfloat16),
            pltpu.VMEM((m_half, k), jnp.bfloat16),
            pltpu.VMEM((m_half, k), jnp.bfloat16),
            pltpu.SemaphoreType.DMA((2, 2)),
            pltpu.SemaphoreType.DMA((2, 2)),
            pltpu.SemaphoreType.DMA((2, 2)),
            pltpu.SemaphoreType.DMA((2, 2)),
        ],
        compiler_params=pltpu.CompilerParams(collective_id=0),
    )(x, w_mat)


# device time: 46210 ns/iter; 1.7433x vs baseline; 1.0016x over previous
import jax
import jax.numpy as jnp
from jax import lax
from jax.experimental import pallas as pl
from jax.experimental.pallas import tpu as pltpu

N_DEV = 4


def kernel(x, w_mat):
    m_per, k = x.shape
    _, n_per = w_mat.shape
    m_half = m_per // 2
    m_seg = m_half // 2

    def body(x_ref, w_ref, out_ref,
             x_bf, from_l, from_r, diag_t, diag_b,
             p1_send, p1_recv, p2_send, p2_recv):
        my_pos = lax.axis_index("i")
        left = (my_pos - 1) % N_DEV
        right = (my_pos + 1) % N_DEV

        barrier_sem = pltpu.get_barrier_semaphore()
        for nbr in [left, right]:
            pl.semaphore_signal(
                barrier_sem, inc=1,
                device_id=(nbr,), device_id_type=pl.DeviceIdType.MESH,
            )
        pl.semaphore_wait(barrier_sem, 2)

        top = pl.ds(0, m_half)
        bot = pl.ds(m_half, m_half)

        x_bf[top, :] = x_ref[top, :].astype(jnp.bfloat16)

        def p1(dir_, seg_idx, rows):
            tgt = right if dir_ == 0 else left
            dst = from_l if dir_ == 0 else from_r
            return pltpu.make_async_remote_copy(
                src_ref=x_bf.at[rows],
                dst_ref=dst.at[rows],
                send_sem=p1_send.at[dir_, seg_idx],
                recv_sem=p1_recv.at[dir_, seg_idx],
                device_id=(tgt,),
                device_id_type=pl.DeviceIdType.MESH,
            )

        p1cw_t = p1(0, 0, top)
        p1cw_b = p1(0, 1, bot)
        p1ccw_b = p1(1, 0, bot)
        p1ccw_t = p1(1, 1, top)

        p1cw_t.start()
        x_bf[bot, :] = x_ref[bot, :].astype(jnp.bfloat16)
        p1ccw_b.start()
        p1cw_b.start()
        p1ccw_t.start()

        def p2(dir_, seg_idx):
            if dir_ == 0:
                rows = pl.ds(seg_idx * m_seg, m_seg)
                return pltpu.make_async_remote_copy(
                    src_ref=from_l.at[rows],
                    dst_ref=diag_t.at[rows],
                    send_sem=p2_send.at[0, seg_idx],
                    recv_sem=p2_recv.at[0, seg_idx],
                    device_id=(right,),
                    device_id_type=pl.DeviceIdType.MESH,
                )
            rows = pl.ds(m_half + seg_idx * m_seg, m_seg)
            drows = pl.ds(seg_idx * m_seg, m_seg)
            return pltpu.make_async_remote_copy(
                src_ref=from_r.at[rows],
                dst_ref=diag_b.at[drows],
                send_sem=p2_send.at[1, seg_idx],
                recv_sem=p2_recv.at[1, seg_idx],
                device_id=(left,),
                device_id_type=pl.DeviceIdType.MESH,
            )

        p2cw = [p2(0, s) for s in range(2)]
        p2ccw = [p2(1, s) for s in range(2)]

        def gemm(rows_src, out_start, rows_n):
            out_ref[pl.ds(out_start, rows_n), :] = jnp.maximum(
                jnp.dot(rows_src.astype(jnp.float32), w_ref[...],
                        preferred_element_type=jnp.float32),
                0.0,
            )

        gemm(x_ref[...], my_pos * m_per, m_per)

        p1cw_t.wait_recv()
        p2cw[0].start()
        p2cw[1].start()
        p1ccw_b.wait_recv()
        p2ccw[0].start()
        p2ccw[1].start()

        gemm(from_l[top, :], left * m_per, m_half)
        gemm(from_r[bot, :], right * m_per + m_half, m_half)

        p1cw_b.wait_recv()
        gemm(from_l[bot, :], left * m_per + m_half, m_half)
        p1ccw_t.wait_recv()
        gemm(from_r[top, :], right * m_per, m_half)

        diag = (my_pos + 2) % N_DEV
        for s in range(2):
            p2cw[s].wait_recv()
            gemm(diag_t[s * m_seg:(s + 1) * m_seg, :],
                 diag * m_per + s * m_seg, m_seg)
            p2ccw[s].wait_recv()
            gemm(diag_b[s * m_seg:(s + 1) * m_seg, :],
                 diag * m_per + m_half + s * m_seg, m_seg)

        for r in (p1cw_t, p1cw_b, p1ccw_b, p1ccw_t,
                  p2cw[0], p2cw[1], p2ccw[0], p2ccw[1]):
            r.wait_send()

    return pl.pallas_call(
        body,
        out_shape=jax.ShapeDtypeStruct((N_DEV * m_per, n_per), jnp.float32),
        in_specs=[
            pl.BlockSpec(memory_space=pltpu.VMEM),
            pl.BlockSpec(memory_space=pltpu.VMEM),
        ],
        out_specs=pl.BlockSpec(memory_space=pltpu.VMEM),
        scratch_shapes=[
            pltpu.VMEM((m_per, k), jnp.bfloat16),
            pltpu.VMEM((m_per, k), jnp.bfloat16),
            pltpu.VMEM((m_per, k), jnp.bfloat16),
            pltpu.VMEM((m_half, k), jnp.bfloat16),
            pltpu.VMEM((m_half, k), jnp.bfloat16),
            pltpu.SemaphoreType.DMA((2, 2)),
            pltpu.SemaphoreType.DMA((2, 2)),
            pltpu.SemaphoreType.DMA((2, 2)),
            pltpu.SemaphoreType.DMA((2, 2)),
        ],
        compiler_params=pltpu.CompilerParams(collective_id=0),
    )(x, w_mat)
